# Initial kernel scaffold; baseline (speedup 1.0000x reference)
import jax
import jax.numpy as jnp
from jax.experimental import pallas as pl
from jax.experimental.pallas import tpu as pltpu

_E = 8
_B = 1024
_C, _H, _W = 3, 32, 32
_D = _C * _H * _W
_HID = 512
_LAT = 128
_CLS = 100
_TEMP = 2.0
_BT = 256
_NB = _B // _BT


def _expert_body(flat_ref, We1_ref, be1_ref, We2_ref, be2_ref, We3_ref, be3_ref,
                 Wd1_ref, bd1_ref, Wd2_ref, bd2_ref, Wd3_ref, bd3_ref,
                 Wx1_ref, bx1_ref, Wx2_ref, bx2_ref, Wx3_ref, bx3_ref,
                 recon_ref, err_ref, eo_ref):
    b = pl.program_id(1)
    flat = flat_ref[...]
    f32 = jnp.float32
    h = jnp.maximum(jnp.dot(flat, We1_ref[0], preferred_element_type=f32) + be1_ref[0], 0.0)
    h = jnp.maximum(jnp.dot(h, We2_ref[0], preferred_element_type=f32) + be2_ref[0], 0.0)
    lat = jnp.dot(h, We3_ref[0], preferred_element_type=f32) + be3_ref[0]
    d = jnp.maximum(jnp.dot(lat, Wd1_ref[0], preferred_element_type=f32) + bd1_ref[0], 0.0)
    d = jnp.maximum(jnp.dot(d, Wd2_ref[0], preferred_element_type=f32) + bd2_ref[0], 0.0)
    recon = jnp.dot(d, Wd3_ref[0], preferred_element_type=f32) + bd3_ref[0]
    recon_ref[0] = recon
    err_ref[0, 0, pl.ds(b * _BT, _BT)] = jnp.mean(jnp.abs(recon - flat), axis=1)
    e1 = jnp.maximum(jnp.dot(lat, Wx1_ref[0], preferred_element_type=f32) + bx1_ref[0], 0.0)
    e1 = jnp.maximum(jnp.dot(e1, Wx2_ref[0], preferred_element_type=f32) + bx2_ref[0], 0.0)
    eo_ref[0] = jnp.dot(e1, Wx3_ref[0], preferred_element_type=f32) + bx3_ref[0]


def _route_body(err_ref, eo_ref, logits_ref, rel_ref, idx_ref, mine_ref, mask_ref):
    errs = err_ref[:, 0, :]
    min_v = errs[0:1, :]
    min_i = jnp.zeros((1, _B), jnp.int32)
    for e in range(1, _E):
        v = errs[e:e + 1, :]
        lt = v < min_v
        min_v = jnp.where(lt, v, min_v)
        min_i = jnp.where(lt, e, min_i)
    z = jnp.exp((min_v - errs) / _TEMP)
    rel_ref[...] = z / jnp.sum(z, axis=0, keepdims=True)
    eids = jax.lax.broadcasted_iota(jnp.int32, (_E, _B), 0)
    m = eids == min_i
    mask_ref[...] = m.astype(jnp.int32)
    idx_ref[...] = min_i
    mine_ref[...] = min_v
    eo = eo_ref[...]
    logits_ref[...] = jnp.sum(jnp.where(m[:, :, None], eo, 0.0), axis=0)


def kernel(x, We1, be1, We2, be2, We3, be3, Wd1, bd1, Wd2, bd2, Wd3, bd3,
           Wx1, bx1, Wx2, bx2, Wx3, bx3):
    flat = x.reshape(_B, _D)
    b3 = lambda b: b.reshape(_E, 1, -1)

    wspec = lambda s: pl.BlockSpec((1,) + s, lambda e, b: (e, 0, 0))
    bspec = lambda n: pl.BlockSpec((1, 1, n), lambda e, b: (e, 0, 0))

    recon, errs, eo = pl.pallas_call(
        _expert_body,
        grid=(_E, _NB),
        in_specs=[
            pl.BlockSpec((_BT, _D), lambda e, b: (b, 0)),
            wspec((_D, _HID)), bspec(_HID),
            wspec((_HID, _HID)), bspec(_HID),
            wspec((_HID, _LAT)), bspec(_LAT),
            wspec((_LAT, _HID)), bspec(_HID),
            wspec((_HID, _HID)), bspec(_HID),
            wspec((_HID, _D)), bspec(_D),
            wspec((_LAT, _HID)), bspec(_HID),
            wspec((_HID, _HID)), bspec(_HID),
            wspec((_HID, _CLS)), bspec(_CLS),
        ],
        out_specs=[
            pl.BlockSpec((1, _BT, _D), lambda e, b: (e, b, 0)),
            pl.BlockSpec((1, 1, _B), lambda e, b: (e, 0, 0)),
            pl.BlockSpec((1, _BT, _CLS), lambda e, b: (e, b, 0)),
        ],
        out_shape=[
            jax.ShapeDtypeStruct((_E, _B, _D), jnp.float32),
            jax.ShapeDtypeStruct((_E, 1, _B), jnp.float32),
            jax.ShapeDtypeStruct((_E, _B, _CLS), jnp.float32),
        ],
    )(flat, We1, b3(be1), We2, b3(be2), We3, b3(be3),
      Wd1, b3(bd1), Wd2, b3(bd2), Wd3, b3(bd3),
      Wx1, b3(bx1), Wx2, b3(bx2), Wx3, b3(bx3))

    logits, rel, idx, mine, mask_i = pl.pallas_call(
        _route_body,
        out_shape=[
            jax.ShapeDtypeStruct((_B, _CLS), jnp.float32),
            jax.ShapeDtypeStruct((_E, _B), jnp.float32),
            jax.ShapeDtypeStruct((1, _B), jnp.int32),
            jax.ShapeDtypeStruct((1, _B), jnp.float32),
            jax.ShapeDtypeStruct((_E, _B), jnp.int32),
        ],
    )(errs, eo)

    reconstructions = recon.reshape(_E, _B, _C, _H, _W)
    return (logits, reconstructions, idx.reshape(_B), mine.reshape(_B),
            rel, mask_i.astype(jnp.bool_))


# speedup vs baseline: 1.0043x; 1.0043x over previous
import jax
import jax.numpy as jnp
from jax.experimental import pallas as pl
from jax.experimental.pallas import tpu as pltpu

_E = 8
_B = 1024
_C, _H, _W = 3, 32, 32
_D = _C * _H * _W
_HID = 512
_LAT = 128
_CLS = 100
_TEMP = 2.0
_BT = 256
_NB = _B // _BT


def _expert_body(flat_ref, We1_ref, be1_ref, We2_ref, be2_ref, We3_ref, be3_ref,
                 Wd1_ref, bd1_ref, Wd2_ref, bd2_ref, Wd3_ref, bd3_ref,
                 Wx1_ref, bx1_ref, Wx2_ref, bx2_ref, Wx3_ref, bx3_ref,
                 recon_ref, err_ref, eo_ref):
    b = pl.program_id(1)
    flat = flat_ref[...]
    f32 = jnp.float32
    h = jnp.maximum(jnp.dot(flat, We1_ref[0], preferred_element_type=f32) + be1_ref[0], 0.0)
    h = jnp.maximum(jnp.dot(h, We2_ref[0], preferred_element_type=f32) + be2_ref[0], 0.0)
    lat = jnp.dot(h, We3_ref[0], preferred_element_type=f32) + be3_ref[0]
    d = jnp.maximum(jnp.dot(lat, Wd1_ref[0], preferred_element_type=f32) + bd1_ref[0], 0.0)
    d = jnp.maximum(jnp.dot(d, Wd2_ref[0], preferred_element_type=f32) + bd2_ref[0], 0.0)
    recon = jnp.dot(d, Wd3_ref[0], preferred_element_type=f32) + bd3_ref[0]
    recon_ref[0] = recon
    err_ref[0, 0, pl.ds(b * _BT, _BT)] = jnp.mean(jnp.abs(recon - flat), axis=1)
    e1 = jnp.maximum(jnp.dot(lat, Wx1_ref[0], preferred_element_type=f32) + bx1_ref[0], 0.0)
    e1 = jnp.maximum(jnp.dot(e1, Wx2_ref[0], preferred_element_type=f32) + bx2_ref[0], 0.0)
    eo_ref[0] = jnp.dot(e1, Wx3_ref[0], preferred_element_type=f32) + bx3_ref[0]


def _route_body(err_ref, errc_ref, eo_ref, logits_ref, rel_ref, idx_ref, mine_ref, mask_ref):
    errs = err_ref[:, 0, :]
    min_v = errs[0:1, :]
    min_i = jnp.zeros((1, _B), jnp.int32)
    for e in range(1, _E):
        v = errs[e:e + 1, :]
        lt = v < min_v
        min_v = jnp.where(lt, v, min_v)
        min_i = jnp.where(lt, e, min_i)
    z = jnp.exp((min_v - errs) / _TEMP)
    rel_ref[...] = z / jnp.sum(z, axis=0, keepdims=True)
    eids = jax.lax.broadcasted_iota(jnp.int32, (_E, _B), 0)
    mask_ref[...] = (eids == min_i).astype(jnp.int32)
    idx_ref[...] = min_i
    mine_ref[...] = min_v
    errc = errc_ref[...]
    min_vc = errc[:, 0:1]
    min_ic = jnp.zeros((_B, 1), jnp.int32)
    for e in range(1, _E):
        v = errc[:, e:e + 1]
        lt = v < min_vc
        min_vc = jnp.where(lt, v, min_vc)
        min_ic = jnp.where(lt, e, min_ic)
    acc = jnp.zeros((_B, _CLS), jnp.float32)
    for e in range(_E):
        acc = acc + eo_ref[e] * (min_ic == e).astype(jnp.float32)
    logits_ref[...] = acc


def kernel(x, We1, be1, We2, be2, We3, be3, Wd1, bd1, Wd2, bd2, Wd3, bd3,
           Wx1, bx1, Wx2, bx2, Wx3, bx3):
    flat = x.reshape(_B, _D)
    b3 = lambda b: b.reshape(_E, 1, -1)

    wspec = lambda s: pl.BlockSpec((1,) + s, lambda e, b: (e, 0, 0))
    bspec = lambda n: pl.BlockSpec((1, 1, n), lambda e, b: (e, 0, 0))

    recon, errs, eo = pl.pallas_call(
        _expert_body,
        grid=(_E, _NB),
        in_specs=[
            pl.BlockSpec((_BT, _D), lambda e, b: (b, 0)),
            wspec((_D, _HID)), bspec(_HID),
            wspec((_HID, _HID)), bspec(_HID),
            wspec((_HID, _LAT)), bspec(_LAT),
            wspec((_LAT, _HID)), bspec(_HID),
            wspec((_HID, _HID)), bspec(_HID),
            wspec((_HID, _D)), bspec(_D),
            wspec((_LAT, _HID)), bspec(_HID),
            wspec((_HID, _HID)), bspec(_HID),
            wspec((_HID, _CLS)), bspec(_CLS),
        ],
        out_specs=[
            pl.BlockSpec((1, _BT, _D), lambda e, b: (e, b, 0)),
            pl.BlockSpec((1, 1, _B), lambda e, b: (e, 0, 0)),
            pl.BlockSpec((1, _BT, _CLS), lambda e, b: (e, b, 0)),
        ],
        out_shape=[
            jax.ShapeDtypeStruct((_E, _B, _D), jnp.float32),
            jax.ShapeDtypeStruct((_E, 1, _B), jnp.float32),
            jax.ShapeDtypeStruct((_E, _B, _CLS), jnp.float32),
        ],
    )(flat, We1, b3(be1), We2, b3(be2), We3, b3(be3),
      Wd1, b3(bd1), Wd2, b3(bd2), Wd3, b3(bd3),
      Wx1, b3(bx1), Wx2, b3(bx2), Wx3, b3(bx3))

    errs_col = jnp.swapaxes(errs.reshape(_E, _B), 0, 1)
    logits, rel, idx, mine, mask_i = pl.pallas_call(
        _route_body,
        out_shape=[
            jax.ShapeDtypeStruct((_B, _CLS), jnp.float32),
            jax.ShapeDtypeStruct((_E, _B), jnp.float32),
            jax.ShapeDtypeStruct((1, _B), jnp.int32),
            jax.ShapeDtypeStruct((1, _B), jnp.float32),
            jax.ShapeDtypeStruct((_E, _B), jnp.int32),
        ],
    )(errs, errs_col, eo)

    reconstructions = recon.reshape(_E, _B, _C, _H, _W)
    return (logits, reconstructions, idx.reshape(_B), mine.reshape(_B),
            rel, mask_i.astype(jnp.bool_))


# speedup vs baseline: 1.0097x; 1.0054x over previous
import jax
import jax.numpy as jnp
from jax.experimental import pallas as pl
from jax.experimental.pallas import tpu as pltpu

_E = 8
_B = 1024
_C, _H, _W = 3, 32, 32
_D = _C * _H * _W
_HID = 512
_LAT = 128
_CLS = 100
_TEMP = 2.0
_BT = 256
_NB = _B // _BT


def _expert_body(flat_hbm, We1_ref, be1_ref, We2_ref, be2_ref, We3_ref, be3_ref,
                 Wd1_ref, bd1_ref, Wd2_ref, bd2_ref, Wd3_ref, bd3_ref,
                 Wx1_ref, bx1_ref, Wx2_ref, bx2_ref, Wx3_ref, bx3_ref,
                 recon_ref, err_ref, eo_ref, flat_scr, dma_sem):
    e = pl.program_id(0)
    b = pl.program_id(1)

    @pl.when((e == 0) & (b == 0))
    def _():
        cp = pltpu.make_async_copy(flat_hbm, flat_scr, dma_sem)
        cp.start()
        cp.wait()

    flat = flat_scr[pl.ds(b * _BT, _BT), :]
    f32 = jnp.float32
    h = jnp.maximum(jnp.dot(flat, We1_ref[0], preferred_element_type=f32) + be1_ref[0], 0.0)
    h = jnp.maximum(jnp.dot(h, We2_ref[0], preferred_element_type=f32) + be2_ref[0], 0.0)
    lat = jnp.dot(h, We3_ref[0], preferred_element_type=f32) + be3_ref[0]
    d = jnp.maximum(jnp.dot(lat, Wd1_ref[0], preferred_element_type=f32) + bd1_ref[0], 0.0)
    d = jnp.maximum(jnp.dot(d, Wd2_ref[0], preferred_element_type=f32) + bd2_ref[0], 0.0)
    recon = jnp.dot(d, Wd3_ref[0], preferred_element_type=f32) + bd3_ref[0]
    recon_ref[0] = recon
    err_ref[0, 0, pl.ds(b * _BT, _BT)] = jnp.mean(jnp.abs(recon - flat), axis=1)
    e1 = jnp.maximum(jnp.dot(lat, Wx1_ref[0], preferred_element_type=f32) + bx1_ref[0], 0.0)
    e1 = jnp.maximum(jnp.dot(e1, Wx2_ref[0], preferred_element_type=f32) + bx2_ref[0], 0.0)
    eo_ref[0] = jnp.dot(e1, Wx3_ref[0], preferred_element_type=f32) + bx3_ref[0]


def _route_body(err_ref, errc_ref, eo_ref, logits_ref, rel_ref, idx_ref, mine_ref, mask_ref):
    errs = err_ref[:, 0, :]
    min_v = errs[0:1, :]
    min_i = jnp.zeros((1, _B), jnp.int32)
    for e in range(1, _E):
        v = errs[e:e + 1, :]
        lt = v < min_v
        min_v = jnp.where(lt, v, min_v)
        min_i = jnp.where(lt, e, min_i)
    z = jnp.exp((min_v - errs) / _TEMP)
    rel_ref[...] = z / jnp.sum(z, axis=0, keepdims=True)
    eids = jax.lax.broadcasted_iota(jnp.int32, (_E, _B), 0)
    mask_ref[...] = (eids == min_i).astype(jnp.int32)
    idx_ref[...] = min_i
    mine_ref[...] = min_v
    errc = errc_ref[...]
    min_vc = errc[:, 0:1]
    min_ic = jnp.zeros((_B, 1), jnp.int32)
    for e in range(1, _E):
        v = errc[:, e:e + 1]
        lt = v < min_vc
        min_vc = jnp.where(lt, v, min_vc)
        min_ic = jnp.where(lt, e, min_ic)
    acc = jnp.zeros((_B, _CLS), jnp.float32)
    for e in range(_E):
        acc = acc + eo_ref[e] * (min_ic == e).astype(jnp.float32)
    logits_ref[...] = acc


def kernel(x, We1, be1, We2, be2, We3, be3, Wd1, bd1, Wd2, bd2, Wd3, bd3,
           Wx1, bx1, Wx2, bx2, Wx3, bx3):
    flat = x.reshape(_B, _D)
    b3 = lambda b: b.reshape(_E, 1, -1)

    wspec = lambda s: pl.BlockSpec((1,) + s, lambda e, b: (e, 0, 0))
    bspec = lambda n: pl.BlockSpec((1, 1, n), lambda e, b: (e, 0, 0))

    recon, errs, eo = pl.pallas_call(
        _expert_body,
        grid=(_E, _NB),
        in_specs=[
            pl.BlockSpec(memory_space=pl.ANY),
            wspec((_D, _HID)), bspec(_HID),
            wspec((_HID, _HID)), bspec(_HID),
            wspec((_HID, _LAT)), bspec(_LAT),
            wspec((_LAT, _HID)), bspec(_HID),
            wspec((_HID, _HID)), bspec(_HID),
            wspec((_HID, _D)), bspec(_D),
            wspec((_LAT, _HID)), bspec(_HID),
            wspec((_HID, _HID)), bspec(_HID),
            wspec((_HID, _CLS)), bspec(_CLS),
        ],
        out_specs=[
            pl.BlockSpec((1, _BT, _D), lambda e, b: (e, b, 0)),
            pl.BlockSpec((1, 1, _B), lambda e, b: (e, 0, 0)),
            pl.BlockSpec((1, _BT, _CLS), lambda e, b: (e, b, 0)),
        ],
        out_shape=[
            jax.ShapeDtypeStruct((_E, _B, _D), jnp.float32),
            jax.ShapeDtypeStruct((_E, 1, _B), jnp.float32),
            jax.ShapeDtypeStruct((_E, _B, _CLS), jnp.float32),
        ],
        scratch_shapes=[
            pltpu.VMEM((_B, _D), jnp.float32),
            pltpu.SemaphoreType.DMA,
        ],
    )(flat, We1, b3(be1), We2, b3(be2), We3, b3(be3),
      Wd1, b3(bd1), Wd2, b3(bd2), Wd3, b3(bd3),
      Wx1, b3(bx1), Wx2, b3(bx2), Wx3, b3(bx3))

    errs_col = jnp.swapaxes(errs.reshape(_E, _B), 0, 1)
    logits, rel, idx, mine, mask_i = pl.pallas_call(
        _route_body,
        out_shape=[
            jax.ShapeDtypeStruct((_B, _CLS), jnp.float32),
            jax.ShapeDtypeStruct((_E, _B), jnp.float32),
            jax.ShapeDtypeStruct((1, _B), jnp.int32),
            jax.ShapeDtypeStruct((1, _B), jnp.float32),
            jax.ShapeDtypeStruct((_E, _B), jnp.int32),
        ],
    )(errs, errs_col, eo)

    reconstructions = recon.reshape(_E, _B, _C, _H, _W)
    return (logits, reconstructions, idx.reshape(_B), mine.reshape(_B),
            rel, mask_i.astype(jnp.bool_))


# speedup vs baseline: 1.1498x; 1.1387x over previous
import jax
import jax.numpy as jnp
from jax.experimental import pallas as pl
from jax.experimental.pallas import tpu as pltpu

_E = 8
_B = 1024
_C, _H, _W = 3, 32, 32
_D = _C * _H * _W
_HID = 512
_LAT = 128
_CLS = 100
_TEMP = 2.0
_BT = 256
_NB = _B // _BT

_W_SHAPES = [(_D, _HID), (_HID, _HID), (_HID, _LAT),
             (_LAT, _HID), (_HID, _HID), (_HID, _D),
             (_LAT, _HID), (_HID, _HID), (_HID, _CLS)]


def _expert_body(flat_hbm, We1_hbm, We2_hbm, We3_hbm, Wd1_hbm, Wd2_hbm, Wd3_hbm,
                 Wx1_hbm, Wx2_hbm, Wx3_hbm,
                 be1_ref, be2_ref, be3_ref, bd1_ref, bd2_ref, bd3_ref,
                 bx1_ref, bx2_ref, bx3_ref,
                 recon_ref, err_ref, eo_ref,
                 flat_scr, w1b, w2b, w3b, w4b, w5b, w6b, w7b, w8b, w9b,
                 wsem, fsem):
    e = pl.program_id(0)
    b = pl.program_id(1)
    hbm = [We1_hbm, We2_hbm, We3_hbm, Wd1_hbm, Wd2_hbm, Wd3_hbm,
           Wx1_hbm, Wx2_hbm, Wx3_hbm]
    buf = [w1b, w2b, w3b, w4b, w5b, w6b, w7b, w8b, w9b]

    def wcopy(i, slot, ei):
        return pltpu.make_async_copy(hbm[i].at[ei], buf[i].at[slot], wsem)

    @pl.when((e == 0) & (b == 0))
    def _():
        cpf = pltpu.make_async_copy(flat_hbm, flat_scr, fsem)
        cpf.start()
        for i in range(9):
            wcopy(i, 0, 0).start()
        for i in range(9):
            wcopy(i, 0, 0).wait()
        cpf.wait()

    @pl.when((b == 0) & (e > 0))
    def _():
        for i in range(9):
            wcopy(i, e % 2, e).wait()

    @pl.when((b == 0) & (e < _E - 1))
    def _():
        for i in range(9):
            wcopy(i, (e + 1) % 2, e + 1).start()

    s = e % 2
    flat = flat_scr[pl.ds(b * _BT, _BT), :]
    f32 = jnp.float32
    h = jnp.maximum(jnp.dot(flat, w1b[s], preferred_element_type=f32) + be1_ref[0], 0.0)
    h = jnp.maximum(jnp.dot(h, w2b[s], preferred_element_type=f32) + be2_ref[0], 0.0)
    lat = jnp.dot(h, w3b[s], preferred_element_type=f32) + be3_ref[0]
    d = jnp.maximum(jnp.dot(lat, w4b[s], preferred_element_type=f32) + bd1_ref[0], 0.0)
    d = jnp.maximum(jnp.dot(d, w5b[s], preferred_element_type=f32) + bd2_ref[0], 0.0)
    recon = jnp.dot(d, w6b[s], preferred_element_type=f32) + bd3_ref[0]
    recon_ref[0] = recon
    err_ref[0, 0, pl.ds(b * _BT, _BT)] = jnp.mean(jnp.abs(recon - flat), axis=1)
    e1 = jnp.maximum(jnp.dot(lat, w7b[s], preferred_element_type=f32) + bx1_ref[0], 0.0)
    e1 = jnp.maximum(jnp.dot(e1, w8b[s], preferred_element_type=f32) + bx2_ref[0], 0.0)
    eo_ref[0] = jnp.dot(e1, w9b[s], preferred_element_type=f32) + bx3_ref[0]


def _route_body(err_ref, errc_ref, eo_ref, logits_ref, rel_ref, idx_ref, mine_ref, mask_ref):
    errs = err_ref[:, 0, :]
    min_v = errs[0:1, :]
    min_i = jnp.zeros((1, _B), jnp.int32)
    for e in range(1, _E):
        v = errs[e:e + 1, :]
        lt = v < min_v
        min_v = jnp.where(lt, v, min_v)
        min_i = jnp.where(lt, e, min_i)
    z = jnp.exp((min_v - errs) / _TEMP)
    rel_ref[...] = z / jnp.sum(z, axis=0, keepdims=True)
    eids = jax.lax.broadcasted_iota(jnp.int32, (_E, _B), 0)
    mask_ref[...] = (eids == min_i).astype(jnp.int32)
    idx_ref[...] = min_i
    mine_ref[...] = min_v
    errc = errc_ref[...]
    min_vc = errc[:, 0:1]
    min_ic = jnp.zeros((_B, 1), jnp.int32)
    for e in range(1, _E):
        v = errc[:, e:e + 1]
        lt = v < min_vc
        min_vc = jnp.where(lt, v, min_vc)
        min_ic = jnp.where(lt, e, min_ic)
    acc = jnp.zeros((_B, _CLS), jnp.float32)
    for e in range(_E):
        acc = acc + eo_ref[e] * (min_ic == e).astype(jnp.float32)
    logits_ref[...] = acc


def kernel(x, We1, be1, We2, be2, We3, be3, Wd1, bd1, Wd2, bd2, Wd3, bd3,
           Wx1, bx1, Wx2, bx2, Wx3, bx3):
    flat = x.reshape(_B, _D)
    b3 = lambda b: b.reshape(_E, 1, -1)

    anyspec = pl.BlockSpec(memory_space=pl.ANY)
    bspec = lambda n: pl.BlockSpec((1, 1, n), lambda e, b: (e, 0, 0))

    recon, errs, eo = pl.pallas_call(
        _expert_body,
        grid=(_E, _NB),
        in_specs=[anyspec] * 10 + [
            bspec(_HID), bspec(_HID), bspec(_LAT),
            bspec(_HID), bspec(_HID), bspec(_D),
            bspec(_HID), bspec(_HID), bspec(_CLS),
        ],
        out_specs=[
            pl.BlockSpec((1, _BT, _D), lambda e, b: (e, b, 0)),
            pl.BlockSpec((1, 1, _B), lambda e, b: (e, 0, 0)),
            pl.BlockSpec((1, _BT, _CLS), lambda e, b: (e, b, 0)),
        ],
        out_shape=[
            jax.ShapeDtypeStruct((_E, _B, _D), jnp.float32),
            jax.ShapeDtypeStruct((_E, 1, _B), jnp.float32),
            jax.ShapeDtypeStruct((_E, _B, _CLS), jnp.float32),
        ],
        scratch_shapes=[
            pltpu.VMEM((_B, _D), jnp.float32),
        ] + [pltpu.VMEM((2,) + s, jnp.float32) for s in _W_SHAPES] + [
            pltpu.SemaphoreType.DMA,
            pltpu.SemaphoreType.DMA,
        ],
    )(flat, We1, We2, We3, Wd1, Wd2, Wd3, Wx1, Wx2, Wx3,
      b3(be1), b3(be2), b3(be3), b3(bd1), b3(bd2), b3(bd3),
      b3(bx1), b3(bx2), b3(bx3))

    errs_col = jnp.swapaxes(errs.reshape(_E, _B), 0, 1)
    logits, rel, idx, mine, mask_i = pl.pallas_call(
        _route_body,
        out_shape=[
            jax.ShapeDtypeStruct((_B, _CLS), jnp.float32),
            jax.ShapeDtypeStruct((_E, _B), jnp.float32),
            jax.ShapeDtypeStruct((1, _B), jnp.int32),
            jax.ShapeDtypeStruct((1, _B), jnp.float32),
            jax.ShapeDtypeStruct((_E, _B), jnp.int32),
        ],
    )(errs, errs_col, eo)

    reconstructions = recon.reshape(_E, _B, _C, _H, _W)
    return (logits, reconstructions, idx.reshape(_B), mine.reshape(_B),
            rel, mask_i.astype(jnp.bool_))


# speedup vs baseline: 1.1512x; 1.0013x over previous
import jax
import jax.numpy as jnp
from jax.experimental import pallas as pl
from jax.experimental.pallas import tpu as pltpu

_E = 8
_B = 1024
_C, _H, _W = 3, 32, 32
_D = _C * _H * _W
_HID = 512
_LAT = 128
_CLS = 100
_TEMP = 2.0
_BT = 256
_NB = _B // _BT

_W_SHAPES = [(_D, _HID), (_HID, _HID), (_HID, _LAT),
             (_LAT, _HID), (_HID, _HID), (_HID, _D),
             (_LAT, _HID), (_HID, _HID), (_HID, _CLS)]


def _expert_body(flat_hbm, We1_hbm, We2_hbm, We3_hbm, Wd1_hbm, Wd2_hbm, Wd3_hbm,
                 Wx1_hbm, Wx2_hbm, Wx3_hbm,
                 be1_ref, be2_ref, be3_ref, bd1_ref, bd2_ref, bd3_ref,
                 bx1_ref, bx2_ref, bx3_ref,
                 recon_ref, err_ref, eo_ref,
                 flat_scr, w1b, w2b, w3b, w4b, w5b, w6b, w7b, w8b, w9b,
                 wsem, fsem):
    e = pl.program_id(0)
    b = pl.program_id(1)
    hbm = [We1_hbm, We2_hbm, We3_hbm, Wd1_hbm, Wd2_hbm, Wd3_hbm,
           Wx1_hbm, Wx2_hbm, Wx3_hbm]
    buf = [w1b, w2b, w3b, w4b, w5b, w6b, w7b, w8b, w9b]

    def wcopy(i, slot, ei):
        return pltpu.make_async_copy(hbm[i].at[ei], buf[i].at[slot], wsem.at[i])

    def fcopy(t):
        return pltpu.make_async_copy(
            flat_hbm.at[pl.ds(t * _BT, _BT), :],
            flat_scr.at[pl.ds(t * _BT, _BT), :], fsem.at[t])

    @pl.when((e == 0) & (b == 0))
    def _():
        for t in range(_NB):
            fcopy(t).start()
        for i in range(9):
            wcopy(i, 0, 0).start()

    @pl.when((e == 0))
    def _():
        fcopy(b).wait()

    @pl.when((b == 0))
    def _():
        for i in range(9):
            wcopy(i, e % 2, e).wait()

    @pl.when((b == 0) & (e < _E - 1))
    def _():
        for i in range(9):
            wcopy(i, (e + 1) % 2, e + 1).start()

    s = e % 2
    flat = flat_scr[pl.ds(b * _BT, _BT), :]
    f32 = jnp.float32
    h = jnp.maximum(jnp.dot(flat, w1b[s], preferred_element_type=f32) + be1_ref[0], 0.0)
    h = jnp.maximum(jnp.dot(h, w2b[s], preferred_element_type=f32) + be2_ref[0], 0.0)
    lat = jnp.dot(h, w3b[s], preferred_element_type=f32) + be3_ref[0]
    d = jnp.maximum(jnp.dot(lat, w4b[s], preferred_element_type=f32) + bd1_ref[0], 0.0)
    d = jnp.maximum(jnp.dot(d, w5b[s], preferred_element_type=f32) + bd2_ref[0], 0.0)
    recon = jnp.dot(d, w6b[s], preferred_element_type=f32) + bd3_ref[0]
    recon_ref[0] = recon
    err_ref[0, 0, pl.ds(b * _BT, _BT)] = jnp.mean(jnp.abs(recon - flat), axis=1)
    e1 = jnp.maximum(jnp.dot(lat, w7b[s], preferred_element_type=f32) + bx1_ref[0], 0.0)
    e1 = jnp.maximum(jnp.dot(e1, w8b[s], preferred_element_type=f32) + bx2_ref[0], 0.0)
    eo_ref[0] = jnp.dot(e1, w9b[s], preferred_element_type=f32) + bx3_ref[0]


def _route_body(err_ref, errc_ref, eo_ref, logits_ref, rel_ref, idx_ref, mine_ref, mask_ref):
    errs = err_ref[:, 0, :]
    min_v = errs[0:1, :]
    min_i = jnp.zeros((1, _B), jnp.int32)
    for e in range(1, _E):
        v = errs[e:e + 1, :]
        lt = v < min_v
        min_v = jnp.where(lt, v, min_v)
        min_i = jnp.where(lt, e, min_i)
    z = jnp.exp((min_v - errs) / _TEMP)
    rel_ref[...] = z / jnp.sum(z, axis=0, keepdims=True)
    eids = jax.lax.broadcasted_iota(jnp.int32, (_E, _B), 0)
    mask_ref[...] = (eids == min_i).astype(jnp.int32)
    idx_ref[...] = min_i
    mine_ref[...] = min_v
    errc = errc_ref[...]
    min_vc = errc[:, 0:1]
    min_ic = jnp.zeros((_B, 1), jnp.int32)
    for e in range(1, _E):
        v = errc[:, e:e + 1]
        lt = v < min_vc
        min_vc = jnp.where(lt, v, min_vc)
        min_ic = jnp.where(lt, e, min_ic)
    acc = jnp.zeros((_B, _CLS), jnp.float32)
    for e in range(_E):
        acc = acc + eo_ref[e] * (min_ic == e).astype(jnp.float32)
    logits_ref[...] = acc


def kernel(x, We1, be1, We2, be2, We3, be3, Wd1, bd1, Wd2, bd2, Wd3, bd3,
           Wx1, bx1, Wx2, bx2, Wx3, bx3):
    flat = x.reshape(_B, _D)
    b3 = lambda b: b.reshape(_E, 1, -1)

    anyspec = pl.BlockSpec(memory_space=pl.ANY)
    bspec = lambda n: pl.BlockSpec((1, 1, n), lambda e, b: (e, 0, 0))

    recon, errs, eo = pl.pallas_call(
        _expert_body,
        grid=(_E, _NB),
        in_specs=[anyspec] * 10 + [
            bspec(_HID), bspec(_HID), bspec(_LAT),
            bspec(_HID), bspec(_HID), bspec(_D),
            bspec(_HID), bspec(_HID), bspec(_CLS),
        ],
        out_specs=[
            pl.BlockSpec((1, _BT, _D), lambda e, b: (e, b, 0)),
            pl.BlockSpec((1, 1, _B), lambda e, b: (e, 0, 0)),
            pl.BlockSpec((1, _BT, _CLS), lambda e, b: (e, b, 0)),
        ],
        out_shape=[
            jax.ShapeDtypeStruct((_E, _B, _D), jnp.float32),
            jax.ShapeDtypeStruct((_E, 1, _B), jnp.float32),
            jax.ShapeDtypeStruct((_E, _B, _CLS), jnp.float32),
        ],
        scratch_shapes=[
            pltpu.VMEM((_B, _D), jnp.float32),
        ] + [pltpu.VMEM((2,) + s, jnp.float32) for s in _W_SHAPES] + [
            pltpu.SemaphoreType.DMA((9,)),
            pltpu.SemaphoreType.DMA((_NB,)),
        ],
    )(flat, We1, We2, We3, Wd1, Wd2, Wd3, Wx1, Wx2, Wx3,
      b3(be1), b3(be2), b3(be3), b3(bd1), b3(bd2), b3(bd3),
      b3(bx1), b3(bx2), b3(bx3))

    errs_col = jnp.swapaxes(errs.reshape(_E, _B), 0, 1)
    logits, rel, idx, mine, mask_i = pl.pallas_call(
        _route_body,
        out_shape=[
            jax.ShapeDtypeStruct((_B, _CLS), jnp.float32),
            jax.ShapeDtypeStruct((_E, _B), jnp.float32),
            jax.ShapeDtypeStruct((1, _B), jnp.int32),
            jax.ShapeDtypeStruct((1, _B), jnp.float32),
            jax.ShapeDtypeStruct((_E, _B), jnp.int32),
        ],
    )(errs, errs_col, eo)

    reconstructions = recon.reshape(_E, _B, _C, _H, _W)
    return (logits, reconstructions, idx.reshape(_B), mine.reshape(_B),
            rel, mask_i.astype(jnp.bool_))
